# baseline (speedup 1.0000x reference)
import functools

import jax
import jax.numpy as jnp
from jax import lax
from jax.experimental import pallas as pl
from jax.experimental.pallas import tpu as pltpu
from jax.experimental.pallas import tpu_sc as plsc

_NUM_WORKERS = 32
_LANES = 16


@functools.partial(jax.jit, static_argnames=("seq", "d_model", "batch"))
def _embed(idx_t, table, seq, d_model, batch):
    mesh = plsc.VectorSubcoreMesh(core_axis_name="c", subcore_axis_name="s")
    bpw = batch // _NUM_WORKERS
    ngroups = d_model // _LANES

    @functools.partial(
        pl.kernel,
        mesh=mesh,
        out_type=jax.ShapeDtypeStruct((seq, d_model, batch), jnp.float32),
        scratch_types=[
            pltpu.VMEM((seq, bpw), jnp.int32),
            [pltpu.VMEM((bpw, d_model), jnp.float32) for _ in range(2)],
            pltpu.VMEM((d_model, bpw), jnp.float32),
            [pltpu.SemaphoreType.DMA for _ in range(2)],
            pltpu.SemaphoreType.DMA,
        ],
        compiler_params=pltpu.CompilerParams(
            use_tc_tiling_on_sc=False, needs_layout_passes=False
        ),
    )
    def k(idx_hbm, tab_hbm, out_hbm, idx_v, rv, tv, g_sems, w_sem):
        cid = lax.axis_index("c")
        sid = lax.axis_index("s")
        wid = sid * 2 + cid
        b0 = wid * bpw
        pltpu.sync_copy(idx_hbm.at[:, pl.ds(b0, bpw)], idx_v)

        def gather_desc(s, buf):
            return pltpu.make_async_copy(tab_hbm.at[idx_v.at[s]], rv[buf], g_sems[buf])

        def write_desc(s):
            return pltpu.make_async_copy(
                tv, out_hbm.at[s, :, pl.ds(b0, bpw)], w_sem
            )

        def transpose(buf):
            row_bases = [
                (lax.iota(jnp.int32, _LANES) + j * _LANES) for j in range(bpw // _LANES)
            ]

            def body(d, carry):
                dvec = jnp.full((_LANES,), d, dtype=jnp.int32)
                for j in range(bpw // _LANES):
                    vals = plsc.load_gather(rv[buf], [row_bases[j], dvec])
                    tv[d, pl.ds(j * _LANES, _LANES)] = vals
                return carry

            lax.fori_loop(0, d_model, body, 0)

        gather_desc(0, 0).start()

        def step(s, buf):
            gather_desc(s, buf).wait()

            @pl.when(s + 1 <= seq - 1)
            def _():
                gather_desc(s + 1, 1 - buf).start()

            @pl.when(s >= 1)
            def _():
                write_desc(s - 1).wait()

            transpose(buf)
            write_desc(s).start()

        def pair(g, carry):
            step(2 * g, 0)
            step(2 * g + 1, 1)
            return carry

        lax.fori_loop(0, seq // 2, pair, 0)
        write_desc(seq - 1).wait()

    return k(idx_t, table)


def kernel(input, table):
    b, s = input.shape
    v, d = table.shape
    idx_t = input.T.astype(jnp.int32)
    out3 = _embed(idx_t, table, s, d, b)
    return jnp.transpose(out3, (2, 0, 1))


# speedup vs baseline: 1.1508x; 1.1508x over previous
import functools

import jax
import jax.numpy as jnp
from jax import lax
from jax.experimental import pallas as pl
from jax.experimental.pallas import tpu as pltpu
from jax.experimental.pallas import tpu_sc as plsc

_NUM_WORKERS = 32
_LANES = 16


@functools.partial(jax.jit, static_argnames=("seq", "d_model", "batch"))
def _embed(idx_t, table, seq, d_model, batch):
    mesh = plsc.VectorSubcoreMesh(core_axis_name="c", subcore_axis_name="s")
    bpw = batch // _NUM_WORKERS
    ngroups = d_model // _LANES

    @functools.partial(
        pl.kernel,
        mesh=mesh,
        out_type=jax.ShapeDtypeStruct((seq, d_model, batch), jnp.float32),
        scratch_types=[
            pltpu.VMEM((seq, bpw), jnp.int32),
            [pltpu.VMEM((bpw, d_model), jnp.float32) for _ in range(2)],
            pltpu.VMEM((d_model, bpw), jnp.float32),
            [pltpu.SemaphoreType.DMA for _ in range(2)],
            pltpu.SemaphoreType.DMA,
        ],
        compiler_params=pltpu.CompilerParams(
            use_tc_tiling_on_sc=False, needs_layout_passes=False
        ),
    )
    def k(idx_hbm, tab_hbm, out_hbm, idx_v, rv, tv, g_sems, w_sem):
        cid = lax.axis_index("c")
        sid = lax.axis_index("s")
        wid = sid * 2 + cid
        b0 = wid * bpw
        pltpu.sync_copy(idx_hbm.at[:, pl.ds(b0, bpw)], idx_v)

        def gather_desc(s, buf):
            return pltpu.make_async_copy(tab_hbm.at[idx_v.at[s]], rv[buf], g_sems[buf])

        def write_desc(s):
            return pltpu.make_async_copy(
                tv, out_hbm.at[s, :, pl.ds(b0, bpw)], w_sem
            )

        def transpose(buf):
            biota = lax.iota(jnp.int32, _LANES)
            nj = bpw // _LANES
            depth = 8

            def body(d, carry):
                dvec = jnp.full((_LANES,), d, dtype=jnp.int32)

                def ld(j):
                    return plsc.load_gather(rv[buf], [biota + j * _LANES, dvec])

                vals = {j: ld(j) for j in range(depth)}
                for j in range(nj):
                    if j + depth < nj:
                        vals[j + depth] = ld(j + depth)
                    tv[d, pl.ds(j * _LANES, _LANES)] = vals.pop(j)
                return carry

            lax.fori_loop(0, d_model, body, 0)

        gather_desc(0, 0).start()

        def step(s, buf):
            gather_desc(s, buf).wait()

            @pl.when(s + 1 <= seq - 1)
            def _():
                gather_desc(s + 1, 1 - buf).start()

            @pl.when(s >= 1)
            def _():
                write_desc(s - 1).wait()

            transpose(buf)
            write_desc(s).start()

        def pair(g, carry):
            step(2 * g, 0)
            step(2 * g + 1, 1)
            return carry

        lax.fori_loop(0, seq // 2, pair, 0)
        write_desc(seq - 1).wait()

    return k(idx_t, table)


def kernel(input, table):
    b, s = input.shape
    v, d = table.shape
    idx_t = input.T.astype(jnp.int32)
    out3 = _embed(idx_t, table, s, d, b)
    return jnp.transpose(out3, (2, 0, 1))


# speedup vs baseline: 1.6722x; 1.4531x over previous
import functools

import jax
import jax.numpy as jnp
from jax import lax
from jax.experimental import pallas as pl
from jax.experimental.pallas import tpu as pltpu
from jax.experimental.pallas import tpu_sc as plsc

_NUM_WORKERS = 32
_CHUNK = 128
_NBUF = 4


@functools.partial(jax.jit, static_argnames=("n_chunks", "d_model"))
def _embed(idx3, table, n_chunks, d_model):
    mesh = plsc.VectorSubcoreMesh(core_axis_name="c", subcore_axis_name="s")
    n_total = _NUM_WORKERS * n_chunks * _CHUNK
    assert n_chunks % _NBUF == 0 and n_chunks > _NBUF

    @functools.partial(
        pl.kernel,
        mesh=mesh,
        out_type=jax.ShapeDtypeStruct((n_total, d_model), jnp.float32),
        scratch_types=[
            pltpu.VMEM((n_chunks, _CHUNK), jnp.int32),
            [pltpu.VMEM((_CHUNK, d_model), jnp.float32) for _ in range(_NBUF)],
            [pltpu.SemaphoreType.DMA for _ in range(_NBUF)],
            [pltpu.SemaphoreType.DMA for _ in range(_NBUF)],
        ],
        compiler_params=pltpu.CompilerParams(use_tc_tiling_on_sc=False),
    )
    def k(idx_hbm, tab_hbm, out_hbm, idx_v, rows, g_sems, w_sems):
        cid = lax.axis_index("c")
        sid = lax.axis_index("s")
        wid = sid * 2 + cid
        pltpu.sync_copy(idx_hbm.at[wid], idx_v)
        base = wid * (n_chunks * _CHUNK)

        def gather_desc(c, b):
            return pltpu.make_async_copy(tab_hbm.at[idx_v.at[c]], rows[b], g_sems[b])

        def write_desc(c, b):
            return pltpu.make_async_copy(
                rows[b], out_hbm.at[pl.ds(base + c * _CHUNK, _CHUNK)], w_sems[b]
            )

        for b in range(_NBUF - 1):
            gather_desc(b, b).start()

        def body(g, carry):
            for b in range(_NBUF):
                c = g + b
                bb = (b - 1) % _NBUF
                gather_desc(c, b).wait()
                write_desc(c, b).start()

                @pl.when(c >= 1)
                def _():
                    write_desc(c - 1, bb).wait()

                @pl.when(c + _NBUF - 1 <= n_chunks - 1)
                def _():
                    gather_desc(c + _NBUF - 1, bb).start()

            return carry

        lax.fori_loop(0, n_chunks // _NBUF, lambda g, x: body(g * _NBUF, x), 0)
        write_desc(n_chunks - 1, (n_chunks - 1) % _NBUF).wait()

    return k(idx3, table)


def kernel(input, table):
    b, s = input.shape
    v, d = table.shape
    n = b * s
    assert n % (_NUM_WORKERS * _CHUNK) == 0
    n_chunks = n // (_NUM_WORKERS * _CHUNK)
    idx3 = input.reshape(_NUM_WORKERS, n_chunks, _CHUNK).astype(jnp.int32)
    out = _embed(idx3, table, n_chunks, d)
    return out.reshape(b, s, d)
